# baseline (speedup 1.0000x reference)
import functools

import jax
import jax.numpy as jnp
from jax.experimental import pallas as pl

N_E = 8192
E_DIM = 32
BETA = 0.25
B = 16
HW = 32 * 32
N_PTS = B * HW
BLK = 256
GRID = N_PTS // BLK
BLK_PER_IMG = HW // BLK


def _vq_body(zsq_ref, z_ref, w_ref, wsq_ref, zq_ref, idx_ref, loss_ref):
    zb = z_ref[0]
    w = w_ref[...]
    wsq = wsq_ref[...]
    zsq = zsq_ref[0]

    z16 = zb.astype(jnp.bfloat16)
    w_hi = w.astype(jnp.bfloat16)
    w_lo = (w - w_hi.astype(jnp.float32)).astype(jnp.bfloat16)
    dn = (((1,), (0,)), ((), ()))
    mm = (jax.lax.dot_general(w_hi, z16, dn, preferred_element_type=jnp.float32)
          + jax.lax.dot_general(w_lo, z16, dn, preferred_element_type=jnp.float32))
    d = (zsq + wsq) - 2.0 * mm
    iota = jax.lax.broadcasted_iota(jnp.int32, (N_E, BLK), 0)
    HALF = N_E // 2
    d0, d1 = d[:HALF], d[HALF:]
    i0c, i1c = iota[:HALF], iota[HALF:]
    m0 = jnp.min(d0, axis=0, keepdims=True)
    m1 = jnp.min(d1, axis=0, keepdims=True)
    idx0 = jnp.min(jnp.where(d0 == m0, i0c, N_E), axis=0)
    idx1 = jnp.min(jnp.where(d1 == m1, i1c, N_E), axis=0)
    carry = m0.astype(jnp.bfloat16).astype(jnp.float32)
    take1 = (m1 < carry)[0]
    idx = jnp.where(take1, idx1, idx0)

    oh = (iota == idx[None, :])
    zqT = jax.lax.dot_general(w, oh.astype(jnp.float32),
                              (((0,), (0,)), ((), ())),
                              precision=jax.lax.Precision.HIGHEST)

    zq_ref[...] = (zb + (zqT - zb))[None]
    idx_ref[...] = idx.reshape(1, 1, BLK)

    s = jnp.sum((zqT - zb) ** 2).reshape(1, 1)
    @pl.when(pl.program_id(0) == 0)
    def _init():
        loss_ref[...] = s
    @pl.when(pl.program_id(0) != 0)
    def _acc():
        loss_ref[...] += s


@functools.partial(jax.jit, static_argnames=())
def kernel(z, W):
    zp = jnp.transpose(z, (0, 2, 3, 1))
    z_flat = zp.reshape(-1, E_DIM)
    zsq = jnp.sum(z_flat ** 2, axis=1).reshape(GRID, 1, BLK)
    wsq = jnp.sum(W ** 2, axis=1).reshape(N_E, 1)
    z3 = z.reshape(B, E_DIM, HW)

    zq3, idx3, loss_acc = pl.pallas_call(
        _vq_body,
        grid=(GRID,),
        in_specs=[
            pl.BlockSpec((1, 1, BLK), lambda i: (i, 0, 0)),
            pl.BlockSpec((1, E_DIM, BLK), lambda i: (i // BLK_PER_IMG, 0, i % BLK_PER_IMG)),
            pl.BlockSpec((N_E, E_DIM), lambda i: (0, 0)),
            pl.BlockSpec((N_E, 1), lambda i: (0, 0)),
        ],
        out_specs=[
            pl.BlockSpec((1, E_DIM, BLK), lambda i: (i // BLK_PER_IMG, 0, i % BLK_PER_IMG)),
            pl.BlockSpec((1, 1, BLK), lambda i: (i, 0, 0)),
            pl.BlockSpec((1, 1), lambda i: (0, 0)),
        ],
        out_shape=[
            jax.ShapeDtypeStruct((B, E_DIM, HW), jnp.float32),
            jax.ShapeDtypeStruct((GRID, 1, BLK), jnp.int32),
            jax.ShapeDtypeStruct((1, 1), jnp.float32),
        ],
    )(zsq, z3, W, wsq)

    z_q_out = zq3.reshape(B, E_DIM, 32, 32)
    min_encoding_indices = idx3.reshape(N_PTS)
    loss = loss_acc[0, 0] * ((1.0 + BETA) / (N_PTS * E_DIM))
    z_indices = min_encoding_indices.reshape(B, 1, 32, 32)
    return (z_q_out, loss, min_encoding_indices, z_indices)
